# Initial kernel scaffold; baseline (speedup 1.0000x reference)
import jax
import jax.numpy as jnp
from jax.experimental import pallas as pl


def kernel(x, edge_index, edge_attr, cluster1, edge_index2, edge_attr2, cluster2, batch, W1, a_src1, a_dst1, b1, W2, a_src2, a_dst2, b2, fc1_W, fc1_b, fc2_W, fc2_b):
    raise NotImplementedError("write your pallas kernel here")


# speedup vs baseline: 19.8594x; 19.8594x over previous
import functools

import jax
import jax.numpy as jnp
from jax import lax
from jax.experimental import pallas as pl
from jax.experimental.pallas import tpu as pltpu
from jax.experimental.pallas import tpu_sc as plsc

F32 = jnp.float32

N = 10000
E = 320000
C1 = 2500
E2 = 80000
C2 = 625
B = 16
D_IN = 128

NW = 32
BLK = 80
NBLK1 = 125
NBLK2 = 32
E2PAD = NW * NBLK2 * BLK
NPT1 = 320
NPT2 = 80
C1PAD = 2560
BP1 = 2528
BP2 = 656


def _mesh():
    return plsc.VectorSubcoreMesh(core_axis_name="c", subcore_axis_name="s",
                                  num_cores=2, num_subcores=16)


def _make_sc_conv(Nn, F, nblk, e_real, mask_tail):
    ept = nblk * BLK
    zrows = Nn // 16
    fv = F // 16

    @functools.partial(
        pl.kernel,
        out_type=[
            jax.ShapeDtypeStruct((2, Nn, F), F32),
            jax.ShapeDtypeStruct((2, Nn), F32),
        ],
        mesh=_mesh(),
        compiler_params=pltpu.CompilerParams(needs_layout_passes=False,
                                             use_tc_tiling_on_sc=False),
        scratch_types=[
            pltpu.VMEM((Nn,), F32),
            pltpu.VMEM((Nn,), F32),
            pltpu.VMEM((nblk, BLK), jnp.int32),
            pltpu.VMEM((nblk, BLK), jnp.int32),
            pltpu.VMEM((nblk, BLK), F32),
            pltpu.VMEM((BLK,), F32),
            pltpu.VMEM((BLK, F), F32),
            pltpu.VMEM_SHARED((Nn, F), F32),
            pltpu.VMEM_SHARED((Nn,), F32),
            pltpu.SemaphoreType.DMA,
        ],
    )
    def conv(h_hbm, hs_hbm, hd_hbm, src_hbm, dst_hbm, ew_hbm, zrow_hbm,
             zden_hbm, s_out, den_out, hs_v, hd_v, src_v, dst_v, ew_v,
             abuf, rowbuf, s_sh, den_sh, sem):
        cid = lax.axis_index("c")
        sid = lax.axis_index("s")
        wid = cid * 16 + sid

        pltpu.sync_copy(hs_hbm, hs_v)
        pltpu.sync_copy(hd_hbm, hd_v)
        pltpu.sync_copy(src_hbm.at[wid], src_v)
        pltpu.sync_copy(dst_hbm.at[wid], dst_v)
        pltpu.sync_copy(ew_hbm.at[wid], ew_v)

        pltpu.sync_copy(zrow_hbm, s_sh.at[pl.ds(sid * zrows, zrows)])

        @pl.when(sid == 0)
        def _():
            pltpu.sync_copy(zden_hbm.at[pl.ds(0, Nn)], den_sh)

        plsc.subcore_barrier()

        def blk_body(j, carry):
            pltpu.async_copy(h_hbm.at[src_v.at[j]], rowbuf, sem).wait()
            for c in range(BLK // 16):
                sv = src_v[j, pl.ds(c * 16, 16)]
                dv = dst_v[j, pl.ds(c * 16, 16)]
                w = ew_v[j, pl.ds(c * 16, 16)]
                s = plsc.load_gather(hs_v, [sv])
                d = plsc.load_gather(hd_v, [dv])
                t = s + d
                a = jnp.exp(jnp.where(t >= 0.0, t, 0.2 * t) * w)
                if mask_tail:
                    ids = (wid * ept + j * BLK + c * 16
                           + lax.iota(jnp.int32, 16))
                    a = jnp.where(ids < e_real, a, 0.0)
                abuf[pl.ds(c * 16, 16)] = a
                for lane in range(16):
                    r = c * 16 + lane
                    av = a[lane]
                    for f in range(fv):
                        rowbuf[r, pl.ds(f * 16, 16)] = (
                            rowbuf[r, pl.ds(f * 16, 16)] * av)
            pltpu.sync_copy(rowbuf, s_sh.at[dst_v.at[j]], add=True)
            pltpu.sync_copy(abuf, den_sh.at[dst_v.at[j]], add=True)
            return carry

        lax.fori_loop(0, nblk, blk_body, 0)
        plsc.subcore_barrier()

        @pl.when(sid == 0)
        def _():
            pltpu.sync_copy(s_sh, s_out.at[cid])
            pltpu.sync_copy(den_sh, den_out.at[cid])

    return conv


def _make_sc_pool(Nn, C, F, npt, n_real, bpad):
    fv = F // 16

    @functools.partial(
        pl.kernel,
        out_type=[
            jax.ShapeDtypeStruct((NW, C, F), F32),
            jax.ShapeDtypeStruct((NW, bpad), F32),
        ],
        mesh=_mesh(),
        compiler_params=pltpu.CompilerParams(needs_layout_passes=False,
                                             use_tc_tiling_on_sc=False),
        scratch_types=[
            pltpu.VMEM((npt, F), F32),
            pltpu.VMEM((npt,), jnp.int32),
            pltpu.VMEM((npt,), F32),
            pltpu.VMEM((C + 1, F), F32),
            pltpu.VMEM((bpad,), F32),
        ],
    )
    def pool(h_hbm, clu_hbm, bat_hbm, parts_out, bparts_out,
             rows_v, clu_v, bat_v, acc_v, bacc_v):
        cid = lax.axis_index("c")
        sid = lax.axis_index("s")
        wid = cid * 16 + sid
        base = jnp.minimum(wid * npt, Nn - npt)

        pltpu.sync_copy(h_hbm.at[pl.ds(base, npt)], rows_v)
        pltpu.sync_copy(clu_hbm.at[pl.ds(base, npt)], clu_v)
        pltpu.sync_copy(bat_hbm.at[pl.ds(base, npt)], bat_v)

        def zero_acc(i, carry):
            for f in range(fv):
                acc_v[i, pl.ds(f * 16, 16)] = jnp.zeros((16,), F32)
            return carry

        lax.fori_loop(0, C + 1, zero_acc, 0)

        def zero_bacc(i, carry):
            bacc_v[pl.ds(i * 16, 16)] = jnp.zeros((16,), F32)
            return carry

        lax.fori_loop(0, bpad // 16, zero_bacc, 0)

        lanes = lax.iota(jnp.int32, 16)

        def body(j, carry):
            cch = clu_v[pl.ds(j * 16, 16)]
            bch = bat_v[pl.ds(j * 16, 16)]
            if n_real is not None:
                gids = base + j * 16 + lanes
                cch = jnp.where(gids < n_real, cch, C)
            for lane in range(16):
                c = cch[lane]
                i = j * 16 + lane
                for f in range(fv):
                    cur = acc_v[c, pl.ds(f * 16, 16)]
                    row = rows_v[i, pl.ds(f * 16, 16)]
                    acc_v[c, pl.ds(f * 16, 16)] = jnp.maximum(cur, row)
                bidx = c + lanes
                bcur = plsc.load_gather(bacc_v, [bidx])
                bnew = jnp.where(lanes == 0,
                                 jnp.maximum(bcur, bch[lane]), bcur)
                plsc.store_scatter(bacc_v, [bidx], bnew)
            return carry

        lax.fori_loop(0, npt // 16, body, 0)

        pltpu.sync_copy(acc_v.at[pl.ds(0, C)], parts_out.at[wid])
        pltpu.sync_copy(bacc_v, bparts_out.at[wid])

    return pool


def _tc_h1pre(x, w1, a1s, a1d):
    def body(x_ref, w_ref, as_ref, ad_ref, h_ref, hs_ref, hd_ref):
        h = jnp.dot(x_ref[...], w_ref[...], preferred_element_type=F32)
        h_ref[...] = h
        hs_ref[...] = jnp.dot(h, as_ref[...], preferred_element_type=F32)
        hd_ref[...] = jnp.dot(h, ad_ref[...], preferred_element_type=F32)

    return pl.pallas_call(
        body,
        grid=(10,),
        in_specs=[
            pl.BlockSpec((1000, 128), lambda i: (i, 0)),
            pl.BlockSpec((128, 16), lambda i: (0, 0)),
            pl.BlockSpec((16, 1), lambda i: (0, 0)),
            pl.BlockSpec((16, 1), lambda i: (0, 0)),
        ],
        out_specs=[
            pl.BlockSpec((1000, 16), lambda i: (i, 0)),
            pl.BlockSpec((1000, 1), lambda i: (i, 0)),
            pl.BlockSpec((1000, 1), lambda i: (i, 0)),
        ],
        out_shape=[
            jax.ShapeDtypeStruct((N, 16), F32),
            jax.ShapeDtypeStruct((N, 1), F32),
            jax.ShapeDtypeStruct((N, 1), F32),
        ],
    )(x, w1, a1s, a1d)


def _tc_norm(s, den, b, Nn, F, nblk):
    del nblk

    def body(s_ref, d_ref, b_ref, o_ref):
        ssum = s_ref[0] + s_ref[1]
        dsum = d_ref[0] + d_ref[1]
        o_ref[...] = jnp.maximum(
            ssum / (dsum[:, None] + 1e-16) + b_ref[...], 0.0)

    return pl.pallas_call(
        body,
        grid=(1,),
        in_specs=[
            pl.BlockSpec((2, Nn, F), lambda i: (0, 0, 0)),
            pl.BlockSpec((2, Nn), lambda i: (0, 0)),
            pl.BlockSpec((1, F), lambda i: (0, 0)),
        ],
        out_specs=pl.BlockSpec((Nn, F), lambda i: (0, 0)),
        out_shape=jax.ShapeDtypeStruct((Nn, F), F32),
    )(s, den, b)


def _tc_mid(parts, bparts, w2, a2s, a2d):
    def body(p_ref, bp_ref, w_ref, as_ref, ad_ref,
             h2_ref, hs_ref, hd_ref, bpo_ref):
        xp = jnp.max(p_ref[...], axis=0)
        bp = jnp.clip(jnp.max(bp_ref[...], axis=0)[:C1], 0.0, 15.0)
        h2 = jnp.dot(xp, w_ref[...], preferred_element_type=F32)
        h2_ref[...] = h2
        hs_ref[...] = jnp.dot(h2, as_ref[...], preferred_element_type=F32)
        hd_ref[...] = jnp.dot(h2, ad_ref[...], preferred_element_type=F32)
        bpo_ref[...] = bp[:, None]

    return pl.pallas_call(
        body,
        grid=(1,),
        in_specs=[
            pl.BlockSpec((NW, C1, 16), lambda i: (0, 0, 0)),
            pl.BlockSpec((NW, BP1), lambda i: (0, 0)),
            pl.BlockSpec((16, 32), lambda i: (0, 0)),
            pl.BlockSpec((32, 1), lambda i: (0, 0)),
            pl.BlockSpec((32, 1), lambda i: (0, 0)),
        ],
        out_specs=[
            pl.BlockSpec((C1, 32), lambda i: (0, 0)),
            pl.BlockSpec((C1, 1), lambda i: (0, 0)),
            pl.BlockSpec((C1, 1), lambda i: (0, 0)),
            pl.BlockSpec((C1, 1), lambda i: (0, 0)),
        ],
        out_shape=[
            jax.ShapeDtypeStruct((C1, 32), F32),
            jax.ShapeDtypeStruct((C1, 1), F32),
            jax.ShapeDtypeStruct((C1, 1), F32),
            jax.ShapeDtypeStruct((C1, 1), F32),
        ],
    )(parts, bparts, w2, a2s, a2d)


def _tc_head(parts2, bparts2, fc1_w, fc1_b, fc2_w, fc2_b):
    def body(p_ref, bq_ref, w1_ref, b1_ref, w2_ref, b2_ref, o_ref):
        x2 = jnp.max(p_ref[...], axis=0)
        bq = jnp.clip(jnp.max(bq_ref[...], axis=0)[:C2], 0.0, 15.0)
        iot = lax.broadcasted_iota(jnp.int32, (B, C2), 0).astype(F32)
        oh = (iot == bq[None, :]).astype(F32)
        cnt = jnp.sum(oh, axis=1)
        ssum = jnp.dot(oh, x2, preferred_element_type=F32)
        xm = ssum / jnp.maximum(cnt, 1.0)[:, None]
        h = jnp.maximum(
            jnp.dot(xm, w1_ref[...], preferred_element_type=F32)
            + b1_ref[...], 0.0)
        o_ref[...] = (jnp.dot(h, w2_ref[...], preferred_element_type=F32)
                      + b2_ref[...])

    return pl.pallas_call(
        body,
        grid=(1,),
        in_specs=[
            pl.BlockSpec((NW, C2, 32), lambda i: (0, 0, 0)),
            pl.BlockSpec((NW, BP2), lambda i: (0, 0)),
            pl.BlockSpec((32, 64), lambda i: (0, 0)),
            pl.BlockSpec((1, 64), lambda i: (0, 0)),
            pl.BlockSpec((64, 1), lambda i: (0, 0)),
            pl.BlockSpec((1, 1), lambda i: (0, 0)),
        ],
        out_specs=pl.BlockSpec((B, 1), lambda i: (0, 0)),
        out_shape=jax.ShapeDtypeStruct((B, 1), F32),
    )(parts2, bparts2, fc1_w, fc1_b, fc2_w, fc2_b)


_sc_conv1 = _make_sc_conv(N, 16, NBLK1, E, False)
_sc_conv2 = _make_sc_conv(C1PAD, 32, NBLK2, E2, True)
_sc_pool1 = _make_sc_pool(N, C1, 16, NPT1, None, BP1)
_sc_pool2 = _make_sc_pool(C1PAD, C2, 32, NPT2, C1, BP2)


def kernel(x, edge_index, edge_attr, cluster1, edge_index2, edge_attr2,
           cluster2, batch, W1, a_src1, a_dst1, b1, W2, a_src2, a_dst2, b2,
           fc1_W, fc1_b, fc2_W, fc2_b):
    h, hs2d, hd2d = _tc_h1pre(x, W1, a_src1.reshape(16, 1),
                              a_dst1.reshape(16, 1))
    hs = hs2d.reshape(N)
    hd = hd2d.reshape(N)

    src3 = edge_index[0].reshape(NW, NBLK1, BLK)
    dst3 = edge_index[1].reshape(NW, NBLK1, BLK)
    ew3 = edge_attr.reshape(E).reshape(NW, NBLK1, BLK)
    s1, den1 = _sc_conv1(h, hs, hd, src3, dst3, ew3,
                         jnp.zeros((N // 16, 16), F32),
                         jnp.zeros((N + 8,), F32))
    h1 = _tc_norm(s1, den1, b1.reshape(1, 16), N, 16, 10)

    parts1, bparts1 = _sc_pool1(h1, cluster1, batch.astype(F32))
    h2, hs2_2d, hd2_2d, bp2d = _tc_mid(parts1, bparts1, W2,
                                       a_src2.reshape(32, 1),
                                       a_dst2.reshape(32, 1))
    hs2 = hs2_2d.reshape(C1)
    hd2 = hd2_2d.reshape(C1)
    bp = bp2d.reshape(C1)

    pad = E2PAD - E2
    src2p = jnp.concatenate(
        [edge_index2[0], jnp.zeros((pad,), jnp.int32)]).reshape(
            NW, NBLK2, BLK)
    dst2p = jnp.concatenate(
        [edge_index2[1], jnp.zeros((pad,), jnp.int32)]).reshape(
            NW, NBLK2, BLK)
    ew2p = jnp.concatenate(
        [edge_attr2.reshape(E2), jnp.zeros((pad,), F32)]).reshape(
            NW, NBLK2, BLK)
    h2pad = jnp.concatenate([h2, jnp.zeros((C1PAD - C1, 32), F32)])
    hs2pad = jnp.concatenate([hs2, jnp.zeros((C1PAD - C1,), F32)])
    hd2pad = jnp.concatenate([hd2, jnp.zeros((C1PAD - C1,), F32)])
    s2, den2 = _sc_conv2(h2pad, hs2pad, hd2pad, src2p, dst2p, ew2p,
                         jnp.zeros((C1PAD // 16, 32), F32),
                         jnp.zeros((C1PAD + 8,), F32))
    h2o = _tc_norm(s2, den2, b2.reshape(1, 32), C1PAD, 32, 4)

    clu2p = jnp.concatenate([cluster2, jnp.zeros((C1PAD - C1,), jnp.int32)])
    bpp = jnp.concatenate([bp, jnp.zeros((C1PAD - C1,), F32)])
    parts2, bparts2 = _sc_pool2(h2o, clu2p, bpp)
    return _tc_head(parts2, bparts2, fc1_W, fc1_b.reshape(1, 64),
                    fc2_W, fc2_b.reshape(1, 1))


# speedup vs baseline: 27.0228x; 1.3607x over previous
import functools

import jax
import jax.numpy as jnp
from jax import lax
from jax.experimental import pallas as pl
from jax.experimental.pallas import tpu as pltpu
from jax.experimental.pallas import tpu_sc as plsc

F32 = jnp.float32

N = 10000
E = 320000
C1 = 2500
E2 = 80000
C2 = 625
B = 16
D_IN = 128

NW = 32
BLK = 80
NBLK1 = 125
NBLK2 = 32
E2PAD = NW * NBLK2 * BLK
NPT1 = 320
NPT2 = 80
C1PAD = 2560
BP1 = 2528
BP2 = 656


def _mesh():
    return plsc.VectorSubcoreMesh(core_axis_name="c", subcore_axis_name="s",
                                  num_cores=2, num_subcores=16)


def _make_sc_conv(Nn, F, nblk, e_real, mask_tail):
    ept = nblk * BLK
    zrows = Nn // 16
    fv = F // 16

    @functools.partial(
        pl.kernel,
        out_type=[
            jax.ShapeDtypeStruct((2, Nn, F), F32),
            jax.ShapeDtypeStruct((2, Nn), F32),
        ],
        mesh=_mesh(),
        compiler_params=pltpu.CompilerParams(needs_layout_passes=False,
                                             use_tc_tiling_on_sc=False),
        scratch_types=[
            pltpu.VMEM((Nn,), F32),
            pltpu.VMEM((Nn,), F32),
            pltpu.VMEM((nblk, BLK), jnp.int32),
            pltpu.VMEM((nblk, BLK), jnp.int32),
            pltpu.VMEM((nblk, BLK), F32),
            pltpu.VMEM((BLK,), F32),
            pltpu.VMEM((BLK,), F32),
            pltpu.VMEM((BLK, F), F32),
            pltpu.VMEM((BLK, F), F32),
            pltpu.VMEM_SHARED((Nn, F), F32),
            pltpu.VMEM_SHARED((Nn,), F32),
            pltpu.SemaphoreType.DMA,
            pltpu.SemaphoreType.DMA,
        ],
    )
    def conv(h_hbm, hs_hbm, hd_hbm, src_hbm, dst_hbm, ew_hbm, zrow_hbm,
             zden_hbm, s_out, den_out, hs_v, hd_v, src_v, dst_v, ew_v,
             abuf, abuf2, rowbuf, rowbuf2, s_sh, den_sh, sem, sem2):
        cid = lax.axis_index("c")
        sid = lax.axis_index("s")
        wid = cid * 16 + sid

        pltpu.sync_copy(hs_hbm, hs_v)
        pltpu.sync_copy(hd_hbm, hd_v)
        pltpu.sync_copy(src_hbm.at[wid], src_v)
        pltpu.sync_copy(dst_hbm.at[wid], dst_v)
        pltpu.sync_copy(ew_hbm.at[wid], ew_v)

        pltpu.sync_copy(zrow_hbm, s_sh.at[pl.ds(sid * zrows, zrows)])

        @pl.when(sid == 0)
        def _():
            pltpu.sync_copy(zden_hbm.at[pl.ds(0, Nn)], den_sh)

        plsc.subcore_barrier()

        def start_gather(j, rb, sm):
            pltpu.async_copy(h_hbm.at[src_v.at[j]], rb, sm)

        def wait_gather(j, rb, sm):
            pltpu.make_async_copy(h_hbm.at[src_v.at[j]], rb, sm).wait()

        def compute_a(j, ab):
            for c in range(BLK // 16):
                sv = src_v[j, pl.ds(c * 16, 16)]
                dv = dst_v[j, pl.ds(c * 16, 16)]
                w = ew_v[j, pl.ds(c * 16, 16)]
                s = plsc.load_gather(hs_v, [sv])
                d = plsc.load_gather(hd_v, [dv])
                t = s + d
                a = jnp.exp(jnp.where(t >= 0.0, t, 0.2 * t) * w)
                if mask_tail:
                    ids = (wid * ept + j * BLK + c * 16
                           + lax.iota(jnp.int32, 16))
                    a = jnp.where(ids < e_real, a, 0.0)
                ab[pl.ds(c * 16, 16)] = a

        def scale_scatter(j, rb, ab):
            for c in range(BLK // 16):
                a = ab[pl.ds(c * 16, 16)]
                for lane in range(16):
                    r = c * 16 + lane
                    av = a[lane]
                    for f in range(fv):
                        rb[r, pl.ds(f * 16, 16)] = (
                            rb[r, pl.ds(f * 16, 16)] * av)
            pltpu.sync_copy(rb, s_sh.at[dst_v.at[j]], add=True)
            pltpu.sync_copy(ab, den_sh.at[dst_v.at[j]], add=True)

        start_gather(0, rowbuf, sem)

        def pair_body(k, carry):
            j0 = 2 * k
            j1 = 2 * k + 1
            start_gather(j1, rowbuf2, sem2)
            compute_a(j0, abuf)
            wait_gather(j0, rowbuf, sem)
            scale_scatter(j0, rowbuf, abuf)

            @pl.when(j0 + 2 < nblk)
            def _():
                start_gather(j0 + 2, rowbuf, sem)

            compute_a(j1, abuf2)
            wait_gather(j1, rowbuf2, sem2)
            scale_scatter(j1, rowbuf2, abuf2)
            return carry

        lax.fori_loop(0, nblk // 2, pair_body, 0)
        if nblk % 2:
            jt = nblk - 1
            compute_a(jt, abuf)
            wait_gather(jt, rowbuf, sem)
            scale_scatter(jt, rowbuf, abuf)
        plsc.subcore_barrier()

        @pl.when(sid == 0)
        def _():
            pltpu.sync_copy(s_sh, s_out.at[cid])
            pltpu.sync_copy(den_sh, den_out.at[cid])

    return conv


def _make_sc_pool(Nn, C, F, npt, n_real, bpad):
    fv = F // 16

    @functools.partial(
        pl.kernel,
        out_type=[
            jax.ShapeDtypeStruct((NW, C, F), F32),
            jax.ShapeDtypeStruct((NW, bpad), F32),
        ],
        mesh=_mesh(),
        compiler_params=pltpu.CompilerParams(needs_layout_passes=False,
                                             use_tc_tiling_on_sc=False),
        scratch_types=[
            pltpu.VMEM((npt, F), F32),
            pltpu.VMEM((npt, F), F32),
            pltpu.VMEM((npt,), F32),
            pltpu.VMEM((npt,), F32),
            pltpu.VMEM((F,), F32),
            pltpu.VMEM((npt,), jnp.int32),
            pltpu.VMEM((npt,), F32),
            pltpu.VMEM((C + 1, F), F32),
            pltpu.VMEM((bpad,), F32),
        ],
    )
    def pool(s_hbm, den_hbm, b_hbm, clu_hbm, bat_hbm, parts_out, bparts_out,
             s0_v, s1_v, d0_v, d1_v, b_v, clu_v, bat_v, acc_v, bacc_v):
        cid = lax.axis_index("c")
        sid = lax.axis_index("s")
        wid = cid * 16 + sid
        base = jnp.minimum(wid * npt, Nn - npt)

        pltpu.sync_copy(s_hbm.at[0, pl.ds(base, npt)], s0_v)
        pltpu.sync_copy(s_hbm.at[1, pl.ds(base, npt)], s1_v)
        pltpu.sync_copy(den_hbm.at[0, pl.ds(base, npt)], d0_v)
        pltpu.sync_copy(den_hbm.at[1, pl.ds(base, npt)], d1_v)
        pltpu.sync_copy(b_hbm, b_v)
        pltpu.sync_copy(clu_hbm.at[pl.ds(base, npt)], clu_v)
        pltpu.sync_copy(bat_hbm.at[pl.ds(base, npt)], bat_v)

        def zero_acc(i, carry):
            for f in range(fv):
                acc_v[i, pl.ds(f * 16, 16)] = jnp.zeros((16,), F32)
            return carry

        lax.fori_loop(0, C + 1, zero_acc, 0)

        def zero_bacc(i, carry):
            bacc_v[pl.ds(i * 16, 16)] = jnp.zeros((16,), F32)
            return carry

        lax.fori_loop(0, bpad // 16, zero_bacc, 0)

        lanes = lax.iota(jnp.int32, 16)

        def body(j, carry):
            cch = clu_v[pl.ds(j * 16, 16)]
            bch = bat_v[pl.ds(j * 16, 16)]
            dsum = d0_v[pl.ds(j * 16, 16)] + d1_v[pl.ds(j * 16, 16)]
            inv = 1.0 / (dsum + 1e-16)
            if n_real is not None:
                gids = base + j * 16 + lanes
                cch = jnp.where(gids < n_real, cch, C)
            for lane in range(16):
                c = cch[lane]
                i = j * 16 + lane
                iv = inv[lane]
                for f in range(fv):
                    cur = acc_v[c, pl.ds(f * 16, 16)]
                    ssum = (s0_v[i, pl.ds(f * 16, 16)]
                            + s1_v[i, pl.ds(f * 16, 16)])
                    row = jnp.maximum(ssum * iv + b_v[pl.ds(f * 16, 16)],
                                      0.0)
                    acc_v[c, pl.ds(f * 16, 16)] = jnp.maximum(cur, row)
                bidx = c + lanes
                bcur = plsc.load_gather(bacc_v, [bidx])
                bnew = jnp.where(lanes == 0,
                                 jnp.maximum(bcur, bch[lane]), bcur)
                plsc.store_scatter(bacc_v, [bidx], bnew)
            return carry

        lax.fori_loop(0, npt // 16, body, 0)

        pltpu.sync_copy(acc_v.at[pl.ds(0, C)], parts_out.at[wid])
        pltpu.sync_copy(bacc_v, bparts_out.at[wid])

    return pool


def _tc_h1pre(x, w1, a1s, a1d):
    def body(x_ref, w_ref, as_ref, ad_ref, h_ref, hs_ref, hd_ref):
        h = jnp.dot(x_ref[...], w_ref[...], preferred_element_type=F32)
        h_ref[...] = h
        hs_ref[...] = jnp.dot(h, as_ref[...], preferred_element_type=F32)
        hd_ref[...] = jnp.dot(h, ad_ref[...], preferred_element_type=F32)

    return pl.pallas_call(
        body,
        grid=(10,),
        in_specs=[
            pl.BlockSpec((1000, 128), lambda i: (i, 0)),
            pl.BlockSpec((128, 16), lambda i: (0, 0)),
            pl.BlockSpec((16, 1), lambda i: (0, 0)),
            pl.BlockSpec((16, 1), lambda i: (0, 0)),
        ],
        out_specs=[
            pl.BlockSpec((1000, 16), lambda i: (i, 0)),
            pl.BlockSpec((1000, 1), lambda i: (i, 0)),
            pl.BlockSpec((1000, 1), lambda i: (i, 0)),
        ],
        out_shape=[
            jax.ShapeDtypeStruct((N, 16), F32),
            jax.ShapeDtypeStruct((N, 1), F32),
            jax.ShapeDtypeStruct((N, 1), F32),
        ],
    )(x, w1, a1s, a1d)


def _tc_mid(parts, bparts, w2, a2s, a2d):
    def body(p_ref, bp_ref, w_ref, as_ref, ad_ref,
             h2_ref, hs_ref, hd_ref, bpo_ref):
        xp = jnp.max(p_ref[...], axis=0)
        bp = jnp.clip(jnp.max(bp_ref[...], axis=0)[:C1], 0.0, 15.0)
        h2 = jnp.dot(xp, w_ref[...], preferred_element_type=F32)
        h2_ref[...] = h2
        hs_ref[...] = jnp.dot(h2, as_ref[...], preferred_element_type=F32)
        hd_ref[...] = jnp.dot(h2, ad_ref[...], preferred_element_type=F32)
        bpo_ref[...] = bp[:, None]

    return pl.pallas_call(
        body,
        grid=(1,),
        in_specs=[
            pl.BlockSpec((NW, C1, 16), lambda i: (0, 0, 0)),
            pl.BlockSpec((NW, BP1), lambda i: (0, 0)),
            pl.BlockSpec((16, 32), lambda i: (0, 0)),
            pl.BlockSpec((32, 1), lambda i: (0, 0)),
            pl.BlockSpec((32, 1), lambda i: (0, 0)),
        ],
        out_specs=[
            pl.BlockSpec((C1, 32), lambda i: (0, 0)),
            pl.BlockSpec((C1, 1), lambda i: (0, 0)),
            pl.BlockSpec((C1, 1), lambda i: (0, 0)),
            pl.BlockSpec((C1, 1), lambda i: (0, 0)),
        ],
        out_shape=[
            jax.ShapeDtypeStruct((C1, 32), F32),
            jax.ShapeDtypeStruct((C1, 1), F32),
            jax.ShapeDtypeStruct((C1, 1), F32),
            jax.ShapeDtypeStruct((C1, 1), F32),
        ],
    )(parts, bparts, w2, a2s, a2d)


def _tc_head(parts2, bparts2, fc1_w, fc1_b, fc2_w, fc2_b):
    def body(p_ref, bq_ref, w1_ref, b1_ref, w2_ref, b2_ref, o_ref):
        x2 = jnp.max(p_ref[...], axis=0)
        bq = jnp.clip(jnp.max(bq_ref[...], axis=0)[:C2], 0.0, 15.0)
        iot = lax.broadcasted_iota(jnp.int32, (B, C2), 0).astype(F32)
        oh = (iot == bq[None, :]).astype(F32)
        cnt = jnp.sum(oh, axis=1)
        ssum = jnp.dot(oh, x2, preferred_element_type=F32)
        xm = ssum / jnp.maximum(cnt, 1.0)[:, None]
        h = jnp.maximum(
            jnp.dot(xm, w1_ref[...], preferred_element_type=F32)
            + b1_ref[...], 0.0)
        o_ref[...] = (jnp.dot(h, w2_ref[...], preferred_element_type=F32)
                      + b2_ref[...])

    return pl.pallas_call(
        body,
        grid=(1,),
        in_specs=[
            pl.BlockSpec((NW, C2, 32), lambda i: (0, 0, 0)),
            pl.BlockSpec((NW, BP2), lambda i: (0, 0)),
            pl.BlockSpec((32, 64), lambda i: (0, 0)),
            pl.BlockSpec((1, 64), lambda i: (0, 0)),
            pl.BlockSpec((64, 1), lambda i: (0, 0)),
            pl.BlockSpec((1, 1), lambda i: (0, 0)),
        ],
        out_specs=pl.BlockSpec((B, 1), lambda i: (0, 0)),
        out_shape=jax.ShapeDtypeStruct((B, 1), F32),
    )(parts2, bparts2, fc1_w, fc1_b, fc2_w, fc2_b)


_sc_conv1 = _make_sc_conv(N, 16, NBLK1, E, False)
_sc_conv2 = _make_sc_conv(C1PAD, 32, NBLK2, E2, True)
_sc_pool1 = _make_sc_pool(N, C1, 16, NPT1, None, BP1)
_sc_pool2 = _make_sc_pool(C1PAD, C2, 32, NPT2, C1, BP2)


def kernel(x, edge_index, edge_attr, cluster1, edge_index2, edge_attr2,
           cluster2, batch, W1, a_src1, a_dst1, b1, W2, a_src2, a_dst2, b2,
           fc1_W, fc1_b, fc2_W, fc2_b):
    h, hs2d, hd2d = _tc_h1pre(x, W1, a_src1.reshape(16, 1),
                              a_dst1.reshape(16, 1))
    hs = hs2d.reshape(N)
    hd = hd2d.reshape(N)

    src3 = edge_index[0].reshape(NW, NBLK1, BLK)
    dst3 = edge_index[1].reshape(NW, NBLK1, BLK)
    ew3 = edge_attr.reshape(E).reshape(NW, NBLK1, BLK)
    s1, den1 = _sc_conv1(h, hs, hd, src3, dst3, ew3,
                         jnp.zeros((N // 16, 16), F32),
                         jnp.zeros((N + 8,), F32))

    parts1, bparts1 = _sc_pool1(s1, den1, b1, cluster1, batch.astype(F32))
    h2, hs2_2d, hd2_2d, bp2d = _tc_mid(parts1, bparts1, W2,
                                       a_src2.reshape(32, 1),
                                       a_dst2.reshape(32, 1))
    hs2 = hs2_2d.reshape(C1)
    hd2 = hd2_2d.reshape(C1)
    bp = bp2d.reshape(C1)

    pad = E2PAD - E2
    src2p = jnp.concatenate(
        [edge_index2[0], jnp.zeros((pad,), jnp.int32)]).reshape(
            NW, NBLK2, BLK)
    dst2p = jnp.concatenate(
        [edge_index2[1], jnp.zeros((pad,), jnp.int32)]).reshape(
            NW, NBLK2, BLK)
    ew2p = jnp.concatenate(
        [edge_attr2.reshape(E2), jnp.zeros((pad,), F32)]).reshape(
            NW, NBLK2, BLK)
    h2pad = jnp.concatenate([h2, jnp.zeros((C1PAD - C1, 32), F32)])
    hs2pad = jnp.concatenate([hs2, jnp.zeros((C1PAD - C1,), F32)])
    hd2pad = jnp.concatenate([hd2, jnp.zeros((C1PAD - C1,), F32)])
    s2, den2 = _sc_conv2(h2pad, hs2pad, hd2pad, src2p, dst2p, ew2p,
                         jnp.zeros((C1PAD // 16, 32), F32),
                         jnp.zeros((C1PAD + 8,), F32))

    clu2p = jnp.concatenate([cluster2, jnp.zeros((C1PAD - C1,), jnp.int32)])
    bpp = jnp.concatenate([bp, jnp.zeros((C1PAD - C1,), F32)])
    parts2, bparts2 = _sc_pool2(s2, den2, b2, clu2p, bpp)
    return _tc_head(parts2, bparts2, fc1_W, fc1_b.reshape(1, 64),
                    fc2_W, fc2_b.reshape(1, 1))


# speedup vs baseline: 27.7635x; 1.0274x over previous
import functools

import jax
import jax.numpy as jnp
from jax import lax
from jax.experimental import pallas as pl
from jax.experimental.pallas import tpu as pltpu
from jax.experimental.pallas import tpu_sc as plsc

F32 = jnp.float32

N = 10000
E = 320000
C1 = 2500
E2 = 80000
C2 = 625
B = 16
D_IN = 128

NW = 32
BLK = 80
NBLK1 = 125
NBLK2 = 32
E2PAD = NW * NBLK2 * BLK
NPT1 = 320
NPT2 = 80
C1PAD = 2560
BP1 = 2528
BP2 = 656


def _mesh():
    return plsc.VectorSubcoreMesh(core_axis_name="c", subcore_axis_name="s",
                                  num_cores=2, num_subcores=16)


def _make_sc_conv(Nn, F, nblk, e_real, mask_tail):
    ept = nblk * BLK
    zrows = Nn // 16
    fv = F // 16

    @functools.partial(
        pl.kernel,
        out_type=[
            jax.ShapeDtypeStruct((2, Nn, F), F32),
            jax.ShapeDtypeStruct((2, Nn), F32),
        ],
        mesh=_mesh(),
        compiler_params=pltpu.CompilerParams(needs_layout_passes=False,
                                             use_tc_tiling_on_sc=False),
        scratch_types=[
            pltpu.VMEM((Nn,), F32),
            pltpu.VMEM((Nn,), F32),
            pltpu.VMEM((nblk, BLK), jnp.int32),
            pltpu.VMEM((nblk, BLK), jnp.int32),
            pltpu.VMEM((nblk, BLK), F32),
            pltpu.VMEM((BLK,), F32),
            pltpu.VMEM((BLK,), F32),
            pltpu.VMEM((BLK, F), F32),
            pltpu.VMEM((BLK, F), F32),
            pltpu.VMEM_SHARED((Nn, F), F32),
            pltpu.VMEM_SHARED((Nn,), F32),
            pltpu.SemaphoreType.DMA,
            pltpu.SemaphoreType.DMA,
            pltpu.SemaphoreType.DMA,
            pltpu.SemaphoreType.DMA,
        ],
    )
    def conv(h_hbm, hs_hbm, hd_hbm, src_hbm, dst_hbm, ew_hbm, zrow_hbm,
             zden_hbm, s_out, den_out, hs_v, hd_v, src_v, dst_v, ew_v,
             abuf, abuf2, rowbuf, rowbuf2, s_sh, den_sh, sem, sem2,
             sems, sems2):
        cid = lax.axis_index("c")
        sid = lax.axis_index("s")
        wid = cid * 16 + sid

        pltpu.sync_copy(hs_hbm, hs_v)
        pltpu.sync_copy(hd_hbm, hd_v)
        pltpu.sync_copy(src_hbm.at[wid], src_v)
        pltpu.sync_copy(dst_hbm.at[wid], dst_v)
        pltpu.sync_copy(ew_hbm.at[wid], ew_v)

        pltpu.sync_copy(zrow_hbm, s_sh.at[pl.ds(sid * zrows, zrows)])

        @pl.when(sid == 0)
        def _():
            pltpu.sync_copy(zden_hbm.at[pl.ds(0, Nn)], den_sh)

        plsc.subcore_barrier()

        def start_gather(j, rb, sm):
            pltpu.async_copy(h_hbm.at[src_v.at[j]], rb, sm)

        def wait_gather(j, rb, sm):
            pltpu.make_async_copy(h_hbm.at[src_v.at[j]], rb, sm).wait()

        def compute_a(j, ab):
            for c in range(BLK // 16):
                sv = src_v[j, pl.ds(c * 16, 16)]
                dv = dst_v[j, pl.ds(c * 16, 16)]
                w = ew_v[j, pl.ds(c * 16, 16)]
                s = plsc.load_gather(hs_v, [sv])
                d = plsc.load_gather(hd_v, [dv])
                t = s + d
                a = jnp.exp(jnp.where(t >= 0.0, t, 0.2 * t) * w)
                if mask_tail:
                    ids = (wid * ept + j * BLK + c * 16
                           + lax.iota(jnp.int32, 16))
                    a = jnp.where(ids < e_real, a, 0.0)
                ab[pl.ds(c * 16, 16)] = a

        def scale(rb, ab):
            for c in range(BLK // 16):
                a = ab[pl.ds(c * 16, 16)]
                for lane in range(16):
                    r = c * 16 + lane
                    av = a[lane]
                    for f in range(fv):
                        rb[r, pl.ds(f * 16, 16)] = (
                            rb[r, pl.ds(f * 16, 16)] * av)

        def start_scatter(j, rb, ab, sm):
            pltpu.async_copy(rb, s_sh.at[dst_v.at[j]], sm, add=True)
            pltpu.async_copy(ab, den_sh.at[dst_v.at[j]], sm, add=True)

        def wait_scatter(j, rb, ab, sm):
            pltpu.make_async_copy(rb, s_sh.at[dst_v.at[j]], sm).wait()
            pltpu.make_async_copy(ab, den_sh.at[dst_v.at[j]], sm).wait()

        start_gather(0, rowbuf, sem)

        def pair_body(k, carry):
            j0 = 2 * k
            j1 = 2 * k + 1

            @pl.when(k > 0)
            def _():
                wait_scatter(j1 - 2, rowbuf2, abuf2, sems2)

            start_gather(j1, rowbuf2, sem2)
            compute_a(j0, abuf)
            wait_gather(j0, rowbuf, sem)
            scale(rowbuf, abuf)
            start_scatter(j0, rowbuf, abuf, sems)
            compute_a(j1, abuf2)
            wait_gather(j1, rowbuf2, sem2)
            scale(rowbuf2, abuf2)
            wait_scatter(j0, rowbuf, abuf, sems)

            @pl.when(j0 + 2 < nblk)
            def _():
                start_gather(j0 + 2, rowbuf, sem)

            start_scatter(j1, rowbuf2, abuf2, sems2)
            return carry

        lax.fori_loop(0, nblk // 2, pair_body, 0)
        wait_scatter(nblk - 1 - (nblk % 2), rowbuf2, abuf2, sems2)
        if nblk % 2:
            jt = nblk - 1
            compute_a(jt, abuf)
            wait_gather(jt, rowbuf, sem)
            scale(rowbuf, abuf)
            start_scatter(jt, rowbuf, abuf, sems)
            wait_scatter(jt, rowbuf, abuf, sems)
        plsc.subcore_barrier()

        @pl.when(sid == 0)
        def _():
            pltpu.sync_copy(s_sh, s_out.at[cid])
            pltpu.sync_copy(den_sh, den_out.at[cid])

    return conv


def _make_sc_pool(Nn, C, F, npt, n_real, bpad):
    fv = F // 16

    @functools.partial(
        pl.kernel,
        out_type=[
            jax.ShapeDtypeStruct((NW, C, F), F32),
            jax.ShapeDtypeStruct((NW, bpad), F32),
        ],
        mesh=_mesh(),
        compiler_params=pltpu.CompilerParams(needs_layout_passes=False,
                                             use_tc_tiling_on_sc=False),
        scratch_types=[
            pltpu.VMEM((npt, F), F32),
            pltpu.VMEM((npt, F), F32),
            pltpu.VMEM((npt,), F32),
            pltpu.VMEM((npt,), F32),
            pltpu.VMEM((F,), F32),
            pltpu.VMEM((npt,), jnp.int32),
            pltpu.VMEM((npt,), F32),
            pltpu.VMEM((C + 1, F), F32),
            pltpu.VMEM((bpad,), F32),
        ],
    )
    def pool(s_hbm, den_hbm, b_hbm, clu_hbm, bat_hbm, parts_out, bparts_out,
             s0_v, s1_v, d0_v, d1_v, b_v, clu_v, bat_v, acc_v, bacc_v):
        cid = lax.axis_index("c")
        sid = lax.axis_index("s")
        wid = cid * 16 + sid
        base = jnp.minimum(wid * npt, Nn - npt)

        pltpu.sync_copy(s_hbm.at[0, pl.ds(base, npt)], s0_v)
        pltpu.sync_copy(s_hbm.at[1, pl.ds(base, npt)], s1_v)
        pltpu.sync_copy(den_hbm.at[0, pl.ds(base, npt)], d0_v)
        pltpu.sync_copy(den_hbm.at[1, pl.ds(base, npt)], d1_v)
        pltpu.sync_copy(b_hbm, b_v)
        pltpu.sync_copy(clu_hbm.at[pl.ds(base, npt)], clu_v)
        pltpu.sync_copy(bat_hbm.at[pl.ds(base, npt)], bat_v)

        def zero_acc(i, carry):
            for f in range(fv):
                acc_v[i, pl.ds(f * 16, 16)] = jnp.zeros((16,), F32)
            return carry

        lax.fori_loop(0, C + 1, zero_acc, 0, unroll=8)

        def zero_bacc(i, carry):
            bacc_v[pl.ds(i * 16, 16)] = jnp.zeros((16,), F32)
            return carry

        lax.fori_loop(0, bpad // 16, zero_bacc, 0, unroll=8)

        lanes = lax.iota(jnp.int32, 16)

        def body(j, carry):
            cch = clu_v[pl.ds(j * 16, 16)]
            bch = bat_v[pl.ds(j * 16, 16)]
            dsum = d0_v[pl.ds(j * 16, 16)] + d1_v[pl.ds(j * 16, 16)]
            inv = 1.0 / (dsum + 1e-16)
            if n_real is not None:
                gids = base + j * 16 + lanes
                cch = jnp.where(gids < n_real, cch, C)
            for lane in range(16):
                c = cch[lane]
                i = j * 16 + lane
                iv = inv[lane]
                for f in range(fv):
                    cur = acc_v[c, pl.ds(f * 16, 16)]
                    ssum = (s0_v[i, pl.ds(f * 16, 16)]
                            + s1_v[i, pl.ds(f * 16, 16)])
                    row = jnp.maximum(ssum * iv + b_v[pl.ds(f * 16, 16)],
                                      0.0)
                    acc_v[c, pl.ds(f * 16, 16)] = jnp.maximum(cur, row)
                bidx = c + lanes
                bcur = plsc.load_gather(bacc_v, [bidx])
                bnew = jnp.where(lanes == 0,
                                 jnp.maximum(bcur, bch[lane]), bcur)
                plsc.store_scatter(bacc_v, [bidx], bnew)
            return carry

        lax.fori_loop(0, npt // 16, body, 0)

        pltpu.sync_copy(acc_v.at[pl.ds(0, C)], parts_out.at[wid])
        pltpu.sync_copy(bacc_v, bparts_out.at[wid])

    return pool


def _tc_h1pre(x, w1, a1s, a1d):
    def body(x_ref, w_ref, as_ref, ad_ref, h_ref, hs_ref, hd_ref):
        h = jnp.dot(x_ref[...], w_ref[...], preferred_element_type=F32)
        h_ref[...] = h
        hs_ref[...] = jnp.dot(h, as_ref[...], preferred_element_type=F32)
        hd_ref[...] = jnp.dot(h, ad_ref[...], preferred_element_type=F32)

    return pl.pallas_call(
        body,
        grid=(10,),
        in_specs=[
            pl.BlockSpec((1000, 128), lambda i: (i, 0)),
            pl.BlockSpec((128, 16), lambda i: (0, 0)),
            pl.BlockSpec((16, 1), lambda i: (0, 0)),
            pl.BlockSpec((16, 1), lambda i: (0, 0)),
        ],
        out_specs=[
            pl.BlockSpec((1000, 16), lambda i: (i, 0)),
            pl.BlockSpec((1000, 1), lambda i: (i, 0)),
            pl.BlockSpec((1000, 1), lambda i: (i, 0)),
        ],
        out_shape=[
            jax.ShapeDtypeStruct((N, 16), F32),
            jax.ShapeDtypeStruct((N, 1), F32),
            jax.ShapeDtypeStruct((N, 1), F32),
        ],
    )(x, w1, a1s, a1d)


def _tc_mid(parts, bparts, w2, a2s, a2d):
    def body(p_ref, bp_ref, w_ref, as_ref, ad_ref,
             h2_ref, hs_ref, hd_ref, bpo_ref):
        xp = jnp.max(p_ref[...], axis=0)
        bp = jnp.clip(jnp.max(bp_ref[...], axis=0)[:C1], 0.0, 15.0)
        h2 = jnp.dot(xp, w_ref[...], preferred_element_type=F32)
        h2_ref[...] = h2
        hs_ref[...] = jnp.dot(h2, as_ref[...], preferred_element_type=F32)
        hd_ref[...] = jnp.dot(h2, ad_ref[...], preferred_element_type=F32)
        bpo_ref[...] = bp[:, None]

    return pl.pallas_call(
        body,
        grid=(1,),
        in_specs=[
            pl.BlockSpec((NW, C1, 16), lambda i: (0, 0, 0)),
            pl.BlockSpec((NW, BP1), lambda i: (0, 0)),
            pl.BlockSpec((16, 32), lambda i: (0, 0)),
            pl.BlockSpec((32, 1), lambda i: (0, 0)),
            pl.BlockSpec((32, 1), lambda i: (0, 0)),
        ],
        out_specs=[
            pl.BlockSpec((C1, 32), lambda i: (0, 0)),
            pl.BlockSpec((C1, 1), lambda i: (0, 0)),
            pl.BlockSpec((C1, 1), lambda i: (0, 0)),
            pl.BlockSpec((C1, 1), lambda i: (0, 0)),
        ],
        out_shape=[
            jax.ShapeDtypeStruct((C1, 32), F32),
            jax.ShapeDtypeStruct((C1, 1), F32),
            jax.ShapeDtypeStruct((C1, 1), F32),
            jax.ShapeDtypeStruct((C1, 1), F32),
        ],
    )(parts, bparts, w2, a2s, a2d)


def _tc_head(parts2, bparts2, fc1_w, fc1_b, fc2_w, fc2_b):
    def body(p_ref, bq_ref, w1_ref, b1_ref, w2_ref, b2_ref, o_ref):
        x2 = jnp.max(p_ref[...], axis=0)
        bq = jnp.clip(jnp.max(bq_ref[...], axis=0)[:C2], 0.0, 15.0)
        iot = lax.broadcasted_iota(jnp.int32, (B, C2), 0).astype(F32)
        oh = (iot == bq[None, :]).astype(F32)
        cnt = jnp.sum(oh, axis=1)
        ssum = jnp.dot(oh, x2, preferred_element_type=F32)
        xm = ssum / jnp.maximum(cnt, 1.0)[:, None]
        h = jnp.maximum(
            jnp.dot(xm, w1_ref[...], preferred_element_type=F32)
            + b1_ref[...], 0.0)
        o_ref[...] = (jnp.dot(h, w2_ref[...], preferred_element_type=F32)
                      + b2_ref[...])

    return pl.pallas_call(
        body,
        grid=(1,),
        in_specs=[
            pl.BlockSpec((NW, C2, 32), lambda i: (0, 0, 0)),
            pl.BlockSpec((NW, BP2), lambda i: (0, 0)),
            pl.BlockSpec((32, 64), lambda i: (0, 0)),
            pl.BlockSpec((1, 64), lambda i: (0, 0)),
            pl.BlockSpec((64, 1), lambda i: (0, 0)),
            pl.BlockSpec((1, 1), lambda i: (0, 0)),
        ],
        out_specs=pl.BlockSpec((B, 1), lambda i: (0, 0)),
        out_shape=jax.ShapeDtypeStruct((B, 1), F32),
    )(parts2, bparts2, fc1_w, fc1_b, fc2_w, fc2_b)


_sc_conv1 = _make_sc_conv(N, 16, NBLK1, E, False)
_sc_conv2 = _make_sc_conv(C1PAD, 32, NBLK2, E2, True)
_sc_pool1 = _make_sc_pool(N, C1, 16, NPT1, None, BP1)
_sc_pool2 = _make_sc_pool(C1PAD, C2, 32, NPT2, C1, BP2)


def kernel(x, edge_index, edge_attr, cluster1, edge_index2, edge_attr2,
           cluster2, batch, W1, a_src1, a_dst1, b1, W2, a_src2, a_dst2, b2,
           fc1_W, fc1_b, fc2_W, fc2_b):
    h, hs2d, hd2d = _tc_h1pre(x, W1, a_src1.reshape(16, 1),
                              a_dst1.reshape(16, 1))
    hs = hs2d.reshape(N)
    hd = hd2d.reshape(N)

    src3 = edge_index[0].reshape(NW, NBLK1, BLK)
    dst3 = edge_index[1].reshape(NW, NBLK1, BLK)
    ew3 = edge_attr.reshape(E).reshape(NW, NBLK1, BLK)
    s1, den1 = _sc_conv1(h, hs, hd, src3, dst3, ew3,
                         jnp.zeros((N // 16, 16), F32),
                         jnp.zeros((N + 8,), F32))

    parts1, bparts1 = _sc_pool1(s1, den1, b1, cluster1, batch.astype(F32))
    h2, hs2_2d, hd2_2d, bp2d = _tc_mid(parts1, bparts1, W2,
                                       a_src2.reshape(32, 1),
                                       a_dst2.reshape(32, 1))
    hs2 = hs2_2d.reshape(C1)
    hd2 = hd2_2d.reshape(C1)
    bp = bp2d.reshape(C1)

    pad = E2PAD - E2
    src2p = jnp.concatenate(
        [edge_index2[0], jnp.zeros((pad,), jnp.int32)]).reshape(
            NW, NBLK2, BLK)
    dst2p = jnp.concatenate(
        [edge_index2[1], jnp.zeros((pad,), jnp.int32)]).reshape(
            NW, NBLK2, BLK)
    ew2p = jnp.concatenate(
        [edge_attr2.reshape(E2), jnp.zeros((pad,), F32)]).reshape(
            NW, NBLK2, BLK)
    h2pad = jnp.concatenate([h2, jnp.zeros((C1PAD - C1, 32), F32)])
    hs2pad = jnp.concatenate([hs2, jnp.zeros((C1PAD - C1,), F32)])
    hd2pad = jnp.concatenate([hd2, jnp.zeros((C1PAD - C1,), F32)])
    s2, den2 = _sc_conv2(h2pad, hs2pad, hd2pad, src2p, dst2p, ew2p,
                         jnp.zeros((C1PAD // 16, 32), F32),
                         jnp.zeros((C1PAD + 8,), F32))

    clu2p = jnp.concatenate([cluster2, jnp.zeros((C1PAD - C1,), jnp.int32)])
    bpp = jnp.concatenate([bp, jnp.zeros((C1PAD - C1,), F32)])
    parts2, bparts2 = _sc_pool2(s2, den2, b2, clu2p, bpp)
    return _tc_head(parts2, bparts2, fc1_W, fc1_b.reshape(1, 64),
                    fc2_W, fc2_b.reshape(1, 1))
